# Initial kernel scaffold; baseline (speedup 1.0000x reference)
import jax
import jax.numpy as jnp
from jax.experimental import pallas as pl


def kernel(x, edge_index, edge_attr, batch, W0, b0, W1, b1):
    raise NotImplementedError("write your pallas kernel here")


# speedup vs baseline: 7.0638x; 7.0638x over previous
import functools

import jax
import jax.numpy as jnp
from jax import lax
from jax.experimental import pallas as pl
from jax.experimental.pallas import tpu as pltpu
from jax.experimental.pallas import tpu_sc as plsc

NC = 2
NS = 16
NW = NC * NS
L = 16


def _pick_chunk(epw):
    for k in range(128, 7, -8):
        if epw % k == 0:
            return k
    raise ValueError(f"no valid chunk for {epw}")


def _sc_mesh():
    return plsc.VectorSubcoreMesh(
        core_axis_name="c", subcore_axis_name="s",
        num_cores=NC, num_subcores=NS)


def _sc_deg(dst3, w2, n):
    nw, nch, K = dst3.shape
    epw = w2.shape[1]
    rpt = n // NS
    zc = rpt // 5

    @functools.partial(
        pl.kernel,
        out_type=jax.ShapeDtypeStruct((NC, NS, rpt, L), jnp.float32),
        mesh=_sc_mesh(),
        compiler_params=pltpu.CompilerParams(use_tc_tiling_on_sc=False),
        scratch_types=[
            pltpu.VMEM((nch, K), jnp.int32),
            pltpu.VMEM((epw,), jnp.float32),
            pltpu.VMEM((K, L), jnp.float32),
            pltpu.VMEM((zc, L), jnp.float32),
            pltpu.VMEM_SHARED((n, L), jnp.float32),
        ],
    )
    def k(dst_hbm, w_hbm, out_hbm, dstv, wv, msg, zb, acc):
        cid = lax.axis_index("c")
        sid = lax.axis_index("s")
        wid = cid * NS + sid

        def zrow(i, c):
            zb[i, :] = jnp.zeros((L,), jnp.float32)
            return c
        lax.fori_loop(0, zc, zrow, 0)
        for t in range(5):
            pltpu.sync_copy(zb, acc.at[pl.ds(sid * rpt + t * zc, zc)])
        plsc.subcore_barrier()

        pltpu.sync_copy(dst_hbm.at[wid], dstv)
        pltpu.sync_copy(w_hbm.at[wid], wv)

        def chunk(ck, c):
            base = ck * K
            for g in range(K // L):
                wvec = wv[pl.ds(base + g * L, L)]
                for i2 in range(L):
                    msg[g * L + i2, :] = jnp.full((L,), 1.0, jnp.float32) * wvec[i2]
            pltpu.sync_copy(msg, acc.at[dstv.at[ck]], add=True)
            return c
        lax.fori_loop(0, nch, chunk, 0)
        plsc.subcore_barrier()

        pltpu.sync_copy(acc.at[pl.ds(sid * rpt, rpt)], out_hbm.at[cid, sid])

    return k(dst3, w2)


def _sc_scatter(p, src3, dst3, w2):
    n, d = p.shape
    nw, nch, K = src3.shape
    epw = w2.shape[1]
    rpt = n // NS
    zc = rpt // 5
    nv = d // L

    @functools.partial(
        pl.kernel,
        out_type=jax.ShapeDtypeStruct((NC, NS, rpt, d), jnp.float32),
        mesh=_sc_mesh(),
        compiler_params=pltpu.CompilerParams(use_tc_tiling_on_sc=False),
        scratch_types=[
            pltpu.VMEM((nch, K), jnp.int32),
            pltpu.VMEM((nch, K), jnp.int32),
            pltpu.VMEM((epw,), jnp.float32),
            pltpu.VMEM((K, d), jnp.float32),
            pltpu.VMEM((zc, d), jnp.float32),
            pltpu.VMEM_SHARED((n, d), jnp.float32),
        ],
    )
    def k(p_hbm, src_hbm, dst_hbm, w_hbm, out_hbm,
          srcv, dstv, wv, rows, zb, acc):
        cid = lax.axis_index("c")
        sid = lax.axis_index("s")
        wid = cid * NS + sid

        def zrow(i, c):
            for j in range(nv):
                zb[i, pl.ds(j * L, L)] = jnp.zeros((L,), jnp.float32)
            return c
        lax.fori_loop(0, zc, zrow, 0)
        for t in range(5):
            pltpu.sync_copy(zb, acc.at[pl.ds(sid * rpt + t * zc, zc)])
        plsc.subcore_barrier()

        pltpu.sync_copy(src_hbm.at[wid], srcv)
        pltpu.sync_copy(dst_hbm.at[wid], dstv)
        pltpu.sync_copy(w_hbm.at[wid], wv)

        def chunk(ck, c):
            pltpu.sync_copy(p_hbm.at[srcv.at[ck]], rows)
            base = ck * K

            def grp(g, c2):
                wvec = wv[pl.ds(base + g * L, L)]
                for i2 in range(L):
                    i = g * L + i2
                    ws = wvec[i2]
                    for j in range(nv):
                        sl = pl.ds(j * L, L)
                        rows[i, sl] = rows[i, sl] * ws
                return c2
            lax.fori_loop(0, K // L, grp, 0)
            pltpu.sync_copy(rows, acc.at[dstv.at[ck]], add=True)
            return c
        lax.fori_loop(0, nch, chunk, 0)
        plsc.subcore_barrier()

        pltpu.sync_copy(acc.at[pl.ds(sid * rpt, rpt)], out_hbm.at[cid, sid])

    return k(p, src3, dst3, w2)


_BR = 1000


def _tc_stage1(d0, d1, x, W0):
    n, d = x.shape
    g = n // _BR

    def body(d0r, d1r, xr, wr, hr, pr, dvr):
        deg = d0r[...] + d1r[...] + 2.0
        dv = lax.rsqrt(deg)
        h = jnp.dot(xr[...], wr[...], preferred_element_type=jnp.float32,
                    precision=lax.Precision.HIGHEST)
        hr[...] = h
        pr[...] = h * dv
        dvr[...] = dv

    return pl.pallas_call(
        body,
        grid=(g,),
        in_specs=[
            pl.BlockSpec((_BR, 1), lambda i: (i, 0)),
            pl.BlockSpec((_BR, 1), lambda i: (i, 0)),
            pl.BlockSpec((_BR, d), lambda i: (i, 0)),
            pl.BlockSpec((d, d), lambda i: (0, 0)),
        ],
        out_specs=[
            pl.BlockSpec((_BR, d), lambda i: (i, 0)),
            pl.BlockSpec((_BR, d), lambda i: (i, 0)),
            pl.BlockSpec((_BR, 1), lambda i: (i, 0)),
        ],
        out_shape=[
            jax.ShapeDtypeStruct((n, d), jnp.float32),
            jax.ShapeDtypeStruct((n, d), jnp.float32),
            jax.ShapeDtypeStruct((n, 1), jnp.float32),
        ],
    )(d0, d1, x, W0)


def _tc_stage2(a0, a1, dv, h, b, W):
    n, d = h.shape
    g = n // _BR

    def body(a0r, a1r, dvr, hr, br, wr, h1r, p1r):
        dvb = dvr[...]
        o = dvb * (a0r[...] + a1r[...]) + (2.0 * dvb * dvb) * hr[...] + br[...]
        h1 = jnp.dot(o, wr[...], preferred_element_type=jnp.float32,
                     precision=lax.Precision.HIGHEST)
        h1r[...] = h1
        p1r[...] = h1 * dvb

    return pl.pallas_call(
        body,
        grid=(g,),
        in_specs=[
            pl.BlockSpec((_BR, d), lambda i: (i, 0)),
            pl.BlockSpec((_BR, d), lambda i: (i, 0)),
            pl.BlockSpec((_BR, 1), lambda i: (i, 0)),
            pl.BlockSpec((_BR, d), lambda i: (i, 0)),
            pl.BlockSpec((1, d), lambda i: (0, 0)),
            pl.BlockSpec((d, d), lambda i: (0, 0)),
        ],
        out_specs=[
            pl.BlockSpec((_BR, d), lambda i: (i, 0)),
            pl.BlockSpec((_BR, d), lambda i: (i, 0)),
        ],
        out_shape=[
            jax.ShapeDtypeStruct((n, d), jnp.float32),
            jax.ShapeDtypeStruct((n, d), jnp.float32),
        ],
    )(a0, a1, dv, h, b, W)


def _tc_pool(a0, a1, dv, h, b, batch3, G):
    n, d = h.shape
    g = n // _BR

    def body(a0r, a1r, dvr, hr, br, btr, outr, sums, cnt):
        i = pl.program_id(0)

        @pl.when(i == 0)
        def _():
            sums[...] = jnp.zeros((G, d), jnp.float32)
            cnt[...] = jnp.zeros((G, 1), jnp.float32)

        dvb = dvr[...]
        o = dvb * (a0r[...] + a1r[...]) + (2.0 * dvb * dvb) * hr[...] + br[...]
        bt = btr[...].reshape(1, _BR)
        gi = lax.broadcasted_iota(jnp.int32, (G, _BR), 0)
        oh = jnp.where(gi == bt, 1.0, 0.0).astype(jnp.float32)
        sums[...] += jnp.dot(oh, o, preferred_element_type=jnp.float32,
                             precision=lax.Precision.HIGHEST)
        cnt[...] += jnp.sum(oh, axis=1, keepdims=True)

        @pl.when(i == g - 1)
        def _():
            outr[...] = sums[...] / jnp.maximum(cnt[...], 1.0)

    return pl.pallas_call(
        body,
        grid=(g,),
        in_specs=[
            pl.BlockSpec((_BR, d), lambda i: (i, 0)),
            pl.BlockSpec((_BR, d), lambda i: (i, 0)),
            pl.BlockSpec((_BR, 1), lambda i: (i, 0)),
            pl.BlockSpec((_BR, d), lambda i: (i, 0)),
            pl.BlockSpec((1, d), lambda i: (0, 0)),
            pl.BlockSpec((1, 1, _BR), lambda i: (i, 0, 0)),
        ],
        out_specs=pl.BlockSpec((G, d), lambda i: (0, 0)),
        out_shape=jax.ShapeDtypeStruct((G, d), jnp.float32),
        scratch_shapes=[
            pltpu.VMEM((G, d), jnp.float32),
            pltpu.VMEM((G, 1), jnp.float32),
        ],
    )(a0, a1, dv, h, b, batch3)


def kernel(x, edge_index, edge_attr, batch, W0, b0, W1, b1):
    n, d = x.shape
    e = edge_index.shape[1]
    G = 16
    epw = e // NW
    K = _pick_chunk(epw)
    nch = epw // K

    src3 = edge_index[0].reshape(NW, nch, K)
    dst3 = edge_index[1].reshape(NW, nch, K)
    w2 = edge_attr.reshape(NW, epw)

    degp = _sc_deg(dst3, w2, n).reshape(NC, n, L)
    d0 = degp[0, :, 0:1]
    d1 = degp[1, :, 0:1]

    dh = d // 2

    def _scatter_full(p):
        aL = _sc_scatter(p[:, :dh], src3, dst3, w2).reshape(NC, n, dh)
        aR = _sc_scatter(p[:, dh:], src3, dst3, w2).reshape(NC, n, dh)
        return jnp.concatenate([aL, aR], axis=2)

    h0, p0, dv = _tc_stage1(d0, d1, x, W0)
    acc1 = _scatter_full(p0)
    h1, p1 = _tc_stage2(acc1[0], acc1[1], dv, h0, b0.reshape(1, d), W1)
    acc2 = _scatter_full(p1)

    batch3 = batch.reshape(n // _BR, 1, _BR)
    return _tc_pool(acc2[0], acc2[1], dv, h1, b1.reshape(1, d), batch3, G)


# speedup vs baseline: 10.8340x; 1.5337x over previous
import functools

import jax
import jax.numpy as jnp
from jax import lax
from jax.experimental import pallas as pl
from jax.experimental.pallas import tpu as pltpu
from jax.experimental.pallas import tpu_sc as plsc

NC = 2
NS = 16
NW = NC * NS
L = 16


def _pick_chunk(epw):
    for k in range(128, 7, -8):
        if epw % k == 0:
            return k
    raise ValueError(f"no valid chunk for {epw}")


def _sc_mesh():
    return plsc.VectorSubcoreMesh(
        core_axis_name="c", subcore_axis_name="s",
        num_cores=NC, num_subcores=NS)


def _sc_deg(dst3, w2, n):
    nw, nch, K = dst3.shape
    epw = w2.shape[1]
    rpt = n // NS
    zc = rpt // 5

    @functools.partial(
        pl.kernel,
        out_type=jax.ShapeDtypeStruct((NC, NS, rpt, L), jnp.float32),
        mesh=_sc_mesh(),
        compiler_params=pltpu.CompilerParams(use_tc_tiling_on_sc=False),
        scratch_types=[
            pltpu.VMEM((nch, K), jnp.int32),
            pltpu.VMEM((epw,), jnp.float32),
            pltpu.VMEM((K, L), jnp.float32),
            pltpu.VMEM((zc, L), jnp.float32),
            pltpu.VMEM_SHARED((n, L), jnp.float32),
        ],
    )
    def k(dst_hbm, w_hbm, out_hbm, dstv, wv, msg, zb, acc):
        cid = lax.axis_index("c")
        sid = lax.axis_index("s")
        wid = cid * NS + sid

        def zrow(i, c):
            zb[i, :] = jnp.zeros((L,), jnp.float32)
            return c
        lax.fori_loop(0, zc, zrow, 0)
        for t in range(5):
            pltpu.sync_copy(zb, acc.at[pl.ds(sid * rpt + t * zc, zc)])
        plsc.subcore_barrier()

        pltpu.sync_copy(dst_hbm.at[wid], dstv)
        pltpu.sync_copy(w_hbm.at[wid], wv)

        def chunk(ck, c):
            base = ck * K
            for g in range(K // L):
                wvec = wv[pl.ds(base + g * L, L)]
                for i2 in range(L):
                    msg[g * L + i2, :] = jnp.full((L,), 1.0, jnp.float32) * wvec[i2]
            pltpu.sync_copy(msg, acc.at[dstv.at[ck]], add=True)
            return c
        lax.fori_loop(0, nch, chunk, 0)
        plsc.subcore_barrier()

        pltpu.sync_copy(acc.at[pl.ds(sid * rpt, rpt)], out_hbm.at[cid, sid])

    return k(dst3, w2)


def _sc_scatter2(ph, src2, dst2, w2):
    nc, n, dh = ph.shape
    ns, nch, K = src2.shape
    ept = w2.shape[1]
    rpt = n // NS
    nv = dh // L
    NB = 5
    zr = 25

    @functools.partial(
        pl.kernel,
        out_type=jax.ShapeDtypeStruct((NC, NS, rpt, dh), jnp.float32),
        mesh=_sc_mesh(),
        compiler_params=pltpu.CompilerParams(use_tc_tiling_on_sc=False),
        scratch_types=[
            pltpu.VMEM((nch, K), jnp.int32),
            pltpu.VMEM((nch, K), jnp.int32),
            pltpu.VMEM((ept,), jnp.float32),
            pltpu.VMEM((NB, K, dh), jnp.float32),
            pltpu.VMEM((zr, dh), jnp.float32),
            pltpu.VMEM_SHARED((n, dh), jnp.float32),
            pltpu.SemaphoreType.DMA((NB,)),
        ],
    )
    def k(ph_hbm, src_hbm, dst_hbm, w_hbm, out_hbm,
          srcv, dstv, wv, ring, zb, acc, sems):
        cid = lax.axis_index("c")
        sid = lax.axis_index("s")

        pltpu.sync_copy(src_hbm.at[sid], srcv)
        pltpu.sync_copy(dst_hbm.at[sid], dstv)
        pltpu.sync_copy(w_hbm.at[sid], wv)

        def zrow(i, c):
            for j in range(nv):
                zb[i, pl.ds(j * L, L)] = jnp.zeros((L,), jnp.float32)
            return c
        lax.fori_loop(0, zr, zrow, 0)
        for t in range(rpt // zr):
            pltpu.sync_copy(zb, acc.at[pl.ds(sid * rpt + t * zr, zr)])

        def gather_start(ck, b):
            pltpu.async_copy(ph_hbm.at[cid].at[srcv.at[ck]], ring.at[b],
                             sems.at[b])

        def gather_wait(ck, b):
            pltpu.make_async_copy(ph_hbm.at[cid].at[srcv.at[ck]], ring.at[b],
                                  sems.at[b]).wait()

        for b in range(NB):
            gather_start(b, b)
        plsc.subcore_barrier()

        def outer(g, c):
            for b in range(NB):
                ck = g * NB + b
                gather_wait(ck, b)
                base = ck * K

                def grp(q, c2):
                    wvec = wv[pl.ds(base + q * L, L)]
                    for i2 in range(L):
                        i = q * L + i2
                        ws = wvec[i2]
                        for j in range(nv):
                            sl = pl.ds(j * L, L)
                            ring[b, i, sl] = ring[b, i, sl] * ws
                    return c2
                lax.fori_loop(0, K // L, grp, 0)
                pltpu.sync_copy(ring.at[b], acc.at[dstv.at[ck]], add=True)

                @pl.when(ck + NB < nch)
                def _():
                    gather_start(ck + NB, b)
            return c
        lax.fori_loop(0, nch // NB, outer, 0)
        plsc.subcore_barrier()

        pltpu.sync_copy(acc.at[pl.ds(sid * rpt, rpt)], out_hbm.at[cid, sid])

    return k(ph, src2, dst2, w2)


_BR = 1000


def _tc_stage1(d0, d1, x, W0):
    n, d = x.shape
    g = n // _BR

    def body(d0r, d1r, xr, wr, hr, pr, dvr):
        deg = d0r[...] + d1r[...] + 2.0
        dv = lax.rsqrt(deg)
        h = jnp.dot(xr[...], wr[...], preferred_element_type=jnp.float32,
                    precision=lax.Precision.HIGHEST)
        hr[...] = h
        pr[...] = h * dv
        dvr[...] = dv

    return pl.pallas_call(
        body,
        grid=(g,),
        in_specs=[
            pl.BlockSpec((_BR, 1), lambda i: (i, 0)),
            pl.BlockSpec((_BR, 1), lambda i: (i, 0)),
            pl.BlockSpec((_BR, d), lambda i: (i, 0)),
            pl.BlockSpec((d, d), lambda i: (0, 0)),
        ],
        out_specs=[
            pl.BlockSpec((_BR, d), lambda i: (i, 0)),
            pl.BlockSpec((_BR, d), lambda i: (i, 0)),
            pl.BlockSpec((_BR, 1), lambda i: (i, 0)),
        ],
        out_shape=[
            jax.ShapeDtypeStruct((n, d), jnp.float32),
            jax.ShapeDtypeStruct((n, d), jnp.float32),
            jax.ShapeDtypeStruct((n, 1), jnp.float32),
        ],
    )(d0, d1, x, W0)


def _tc_stage2(a, dv, h, b, W):
    n, d = h.shape
    g = n // _BR

    def body(ar, dvr, hr, br, wr, h1r, p1r):
        dvb = dvr[...]
        o = dvb * ar[...] + (2.0 * dvb * dvb) * hr[...] + br[...]
        h1 = jnp.dot(o, wr[...], preferred_element_type=jnp.float32,
                     precision=lax.Precision.HIGHEST)
        h1r[...] = h1
        p1r[...] = h1 * dvb

    return pl.pallas_call(
        body,
        grid=(g,),
        in_specs=[
            pl.BlockSpec((_BR, d), lambda i: (i, 0)),
            pl.BlockSpec((_BR, 1), lambda i: (i, 0)),
            pl.BlockSpec((_BR, d), lambda i: (i, 0)),
            pl.BlockSpec((1, d), lambda i: (0, 0)),
            pl.BlockSpec((d, d), lambda i: (0, 0)),
        ],
        out_specs=[
            pl.BlockSpec((_BR, d), lambda i: (i, 0)),
            pl.BlockSpec((_BR, d), lambda i: (i, 0)),
        ],
        out_shape=[
            jax.ShapeDtypeStruct((n, d), jnp.float32),
            jax.ShapeDtypeStruct((n, d), jnp.float32),
        ],
    )(a, dv, h, b, W)


def _tc_pool(a, dv, h, b, batch3, G):
    n, d = h.shape
    g = n // _BR

    def body(ar, dvr, hr, br, btr, outr, sums, cnt):
        i = pl.program_id(0)

        @pl.when(i == 0)
        def _():
            sums[...] = jnp.zeros((G, d), jnp.float32)
            cnt[...] = jnp.zeros((G, 1), jnp.float32)

        dvb = dvr[...]
        o = dvb * ar[...] + (2.0 * dvb * dvb) * hr[...] + br[...]
        bt = btr[...].reshape(1, _BR)
        gi = lax.broadcasted_iota(jnp.int32, (G, _BR), 0)
        oh = jnp.where(gi == bt, 1.0, 0.0).astype(jnp.float32)
        sums[...] += jnp.dot(oh, o, preferred_element_type=jnp.float32,
                             precision=lax.Precision.HIGHEST)
        cnt[...] += jnp.sum(oh, axis=1, keepdims=True)

        @pl.when(i == g - 1)
        def _():
            outr[...] = sums[...] / jnp.maximum(cnt[...], 1.0)

    return pl.pallas_call(
        body,
        grid=(g,),
        in_specs=[
            pl.BlockSpec((_BR, d), lambda i: (i, 0)),
            pl.BlockSpec((_BR, 1), lambda i: (i, 0)),
            pl.BlockSpec((_BR, d), lambda i: (i, 0)),
            pl.BlockSpec((1, d), lambda i: (0, 0)),
            pl.BlockSpec((1, 1, _BR), lambda i: (i, 0, 0)),
        ],
        out_specs=pl.BlockSpec((G, d), lambda i: (0, 0)),
        out_shape=jax.ShapeDtypeStruct((G, d), jnp.float32),
        scratch_shapes=[
            pltpu.VMEM((G, d), jnp.float32),
            pltpu.VMEM((G, 1), jnp.float32),
        ],
    )(a, dv, h, b, batch3)


def kernel(x, edge_index, edge_attr, batch, W0, b0, W1, b1):
    n, d = x.shape
    e = edge_index.shape[1]
    G = 16
    epw = e // NW
    K = _pick_chunk(epw)
    nch = epw // K

    src3 = edge_index[0].reshape(NW, nch, K)
    dst3 = edge_index[1].reshape(NW, nch, K)
    wnw = edge_attr.reshape(NW, epw)

    degp = _sc_deg(dst3, wnw, n).reshape(NC, n, L)
    d0 = degp[0, :, 0:1]
    d1 = degp[1, :, 0:1]

    dh = d // 2
    ept = e // NS
    nch2 = ept // K
    src2 = edge_index[0].reshape(NS, nch2, K)
    dst2 = edge_index[1].reshape(NS, nch2, K)
    w2 = edge_attr.reshape(NS, ept)

    def _scatter_full(p):
        ph = jnp.stack([p[:, :dh], p[:, dh:]], axis=0)
        a = _sc_scatter2(ph, src2, dst2, w2)
        return jnp.concatenate(
            [a[0].reshape(n, dh), a[1].reshape(n, dh)], axis=1)

    h0, p0, dv = _tc_stage1(d0, d1, x, W0)
    acc1 = _scatter_full(p0)
    h1, p1 = _tc_stage2(acc1, dv, h0, b0.reshape(1, d), W1)
    acc2 = _scatter_full(p1)

    batch3 = batch.reshape(n // _BR, 1, _BR)
    return _tc_pool(acc2, dv, h1, b1.reshape(1, d), batch3, G)


# speedup vs baseline: 11.5171x; 1.0630x over previous
import functools

import jax
import jax.numpy as jnp
from jax import lax
from jax.experimental import pallas as pl
from jax.experimental.pallas import tpu as pltpu
from jax.experimental.pallas import tpu_sc as plsc

NC = 2
NS = 16
NW = NC * NS
L = 16


def _pick_chunk(epw):
    for k in range(128, 7, -8):
        if epw % k == 0:
            return k
    raise ValueError(f"no valid chunk for {epw}")


def _sc_mesh():
    return plsc.VectorSubcoreMesh(
        core_axis_name="c", subcore_axis_name="s",
        num_cores=NC, num_subcores=NS)


def _sc_deg(dst3, w2, n):
    nw, nch, K = dst3.shape
    epw = w2.shape[1]
    rpt = n // NS
    zc = rpt // 5

    @functools.partial(
        pl.kernel,
        out_type=jax.ShapeDtypeStruct((NC, NS, rpt, L), jnp.float32),
        mesh=_sc_mesh(),
        compiler_params=pltpu.CompilerParams(use_tc_tiling_on_sc=False),
        scratch_types=[
            pltpu.VMEM((nch, K), jnp.int32),
            pltpu.VMEM((epw,), jnp.float32),
            pltpu.VMEM((K, L), jnp.float32),
            pltpu.VMEM((zc, L), jnp.float32),
            pltpu.VMEM_SHARED((n, L), jnp.float32),
        ],
    )
    def k(dst_hbm, w_hbm, out_hbm, dstv, wv, msg, zb, acc):
        cid = lax.axis_index("c")
        sid = lax.axis_index("s")
        wid = cid * NS + sid

        def zrow(i, c):
            zb[i, :] = jnp.zeros((L,), jnp.float32)
            return c
        lax.fori_loop(0, zc, zrow, 0)
        for t in range(5):
            pltpu.sync_copy(zb, acc.at[pl.ds(sid * rpt + t * zc, zc)])
        plsc.subcore_barrier()

        pltpu.sync_copy(dst_hbm.at[wid], dstv)
        pltpu.sync_copy(w_hbm.at[wid], wv)

        def chunk(ck, c):
            base = ck * K
            for g in range(K // L):
                wvec = wv[pl.ds(base + g * L, L)]
                for i2 in range(L):
                    msg[g * L + i2, :] = jnp.full((L,), 1.0, jnp.float32) * wvec[i2]
            pltpu.sync_copy(msg, acc.at[dstv.at[ck]], add=True)
            return c
        lax.fori_loop(0, nch, chunk, 0)
        plsc.subcore_barrier()

        pltpu.sync_copy(acc.at[pl.ds(sid * rpt, rpt)], out_hbm.at[cid, sid])

    return k(dst3, w2)


def _sc_scatter2(ph, src2, dst2, w2):
    nc, n, dh = ph.shape
    ns, nch, K = src2.shape
    ept = w2.shape[1]
    rpt = n // NS
    nv = dh // L
    NB = 5
    zr = 25

    @functools.partial(
        pl.kernel,
        out_type=jax.ShapeDtypeStruct((NC, NS, rpt, dh), jnp.float32),
        mesh=_sc_mesh(),
        compiler_params=pltpu.CompilerParams(use_tc_tiling_on_sc=False),
        scratch_types=[
            pltpu.VMEM((nch, K), jnp.int32),
            pltpu.VMEM((nch, K), jnp.int32),
            pltpu.VMEM((ept,), jnp.float32),
            pltpu.VMEM((NB, K, dh), jnp.float32),
            pltpu.VMEM((zr, dh), jnp.float32),
            pltpu.VMEM_SHARED((n, dh), jnp.float32),
            pltpu.SemaphoreType.DMA((NB,)),
        ],
    )
    def k(ph_hbm, src_hbm, dst_hbm, w_hbm, out_hbm,
          srcv, dstv, wv, ring, zb, acc, sems):
        cid = lax.axis_index("c")
        sid = lax.axis_index("s")

        pltpu.sync_copy(src_hbm.at[sid], srcv)
        pltpu.sync_copy(dst_hbm.at[sid], dstv)
        pltpu.sync_copy(w_hbm.at[sid], wv)

        def zrow(i, c):
            for j in range(nv):
                zb[i, pl.ds(j * L, L)] = jnp.zeros((L,), jnp.float32)
            return c
        lax.fori_loop(0, zr, zrow, 0)
        for t in range(rpt // zr):
            pltpu.sync_copy(zb, acc.at[pl.ds(sid * rpt + t * zr, zr)])

        def gather_start(ck, b):
            pltpu.async_copy(ph_hbm.at[cid].at[srcv.at[ck]], ring.at[b],
                             sems.at[b])

        def gather_wait(ck, b):
            pltpu.make_async_copy(ph_hbm.at[cid].at[srcv.at[ck]], ring.at[b],
                                  sems.at[b]).wait()

        for b in range(NB):
            gather_start(b, b)
        plsc.subcore_barrier()

        def outer(g, c):
            for b in range(NB):
                ck = g * NB + b
                gather_wait(ck, b)
                base = ck * K

                def grp(q, c2):
                    wvec = wv[pl.ds(base + q * L, L)]
                    for i2 in range(L):
                        i = q * L + i2
                        ws = wvec[i2]
                        for j in range(nv):
                            sl = pl.ds(j * L, L)
                            ring[b, i, sl] = ring[b, i, sl] * ws
                    return c2
                lax.fori_loop(0, K // L, grp, 0)
                pltpu.sync_copy(ring.at[b], acc.at[dstv.at[ck]], add=True)

                @pl.when(ck + NB < nch)
                def _():
                    gather_start(ck + NB, b)
            return c
        lax.fori_loop(0, nch // NB, outer, 0)
        plsc.subcore_barrier()

        pltpu.sync_copy(acc.at[pl.ds(sid * rpt, rpt)], out_hbm.at[cid, sid])

    return k(ph, src2, dst2, w2)


_BR = 1000


def _tc_stage1(d0, d1, x, W0):
    n, d = x.shape
    dh = d // 2
    g = n // _BR

    def body(d0r, d1r, xr, wr, hr, phr, dvr):
        deg = d0r[...] + d1r[...] + 2.0
        dv = lax.rsqrt(deg)
        h = jnp.dot(xr[...], wr[...], preferred_element_type=jnp.float32,
                    precision=lax.Precision.HIGHEST)
        hr[...] = h
        p = h * dv
        phr[0, :, :] = p[:, :dh]
        phr[1, :, :] = p[:, dh:]
        dvr[...] = dv

    return pl.pallas_call(
        body,
        grid=(g,),
        in_specs=[
            pl.BlockSpec((_BR, 1), lambda i: (i, 0)),
            pl.BlockSpec((_BR, 1), lambda i: (i, 0)),
            pl.BlockSpec((_BR, d), lambda i: (i, 0)),
            pl.BlockSpec((d, d), lambda i: (0, 0)),
        ],
        out_specs=[
            pl.BlockSpec((_BR, d), lambda i: (i, 0)),
            pl.BlockSpec((2, _BR, dh), lambda i: (0, i, 0)),
            pl.BlockSpec((_BR, 1), lambda i: (i, 0)),
        ],
        out_shape=[
            jax.ShapeDtypeStruct((n, d), jnp.float32),
            jax.ShapeDtypeStruct((2, n, dh), jnp.float32),
            jax.ShapeDtypeStruct((n, 1), jnp.float32),
        ],
    )(d0, d1, x, W0)


def _tc_stage2(a2, dv, h, b, W):
    n, d = h.shape
    dh = d // 2
    g = n // _BR

    def body(ar, dvr, hr, br, wr, h1r, p1r):
        dvb = dvr[...]
        af = jnp.concatenate([ar[0], ar[1]], axis=1)
        o = dvb * af + (2.0 * dvb * dvb) * hr[...] + br[...]
        h1 = jnp.dot(o, wr[...], preferred_element_type=jnp.float32,
                     precision=lax.Precision.HIGHEST)
        h1r[...] = h1
        p1 = h1 * dvb
        p1r[0, :, :] = p1[:, :dh]
        p1r[1, :, :] = p1[:, dh:]

    return pl.pallas_call(
        body,
        grid=(g,),
        in_specs=[
            pl.BlockSpec((2, _BR, dh), lambda i: (0, i, 0)),
            pl.BlockSpec((_BR, 1), lambda i: (i, 0)),
            pl.BlockSpec((_BR, d), lambda i: (i, 0)),
            pl.BlockSpec((1, d), lambda i: (0, 0)),
            pl.BlockSpec((d, d), lambda i: (0, 0)),
        ],
        out_specs=[
            pl.BlockSpec((_BR, d), lambda i: (i, 0)),
            pl.BlockSpec((2, _BR, dh), lambda i: (0, i, 0)),
        ],
        out_shape=[
            jax.ShapeDtypeStruct((n, d), jnp.float32),
            jax.ShapeDtypeStruct((2, n, dh), jnp.float32),
        ],
    )(a2, dv, h, b, W)


def _tc_pool(a2, dv, h, b, batch3, G):
    n, d = h.shape
    dh = d // 2
    g = n // _BR

    def body(ar, dvr, hr, br, btr, outr, sums, cnt):
        i = pl.program_id(0)

        @pl.when(i == 0)
        def _():
            sums[...] = jnp.zeros((G, d), jnp.float32)
            cnt[...] = jnp.zeros((G, 1), jnp.float32)

        dvb = dvr[...]
        af = jnp.concatenate([ar[0], ar[1]], axis=1)
        o = dvb * af + (2.0 * dvb * dvb) * hr[...] + br[...]
        bt = btr[...].reshape(1, _BR)
        gi = lax.broadcasted_iota(jnp.int32, (G, _BR), 0)
        oh = jnp.where(gi == bt, 1.0, 0.0).astype(jnp.float32)
        sums[...] += jnp.dot(oh, o, preferred_element_type=jnp.float32,
                             precision=lax.Precision.HIGHEST)
        cnt[...] += jnp.sum(oh, axis=1, keepdims=True)

        @pl.when(i == g - 1)
        def _():
            outr[...] = sums[...] / jnp.maximum(cnt[...], 1.0)

    return pl.pallas_call(
        body,
        grid=(g,),
        in_specs=[
            pl.BlockSpec((2, _BR, dh), lambda i: (0, i, 0)),
            pl.BlockSpec((_BR, 1), lambda i: (i, 0)),
            pl.BlockSpec((_BR, d), lambda i: (i, 0)),
            pl.BlockSpec((1, d), lambda i: (0, 0)),
            pl.BlockSpec((1, 1, _BR), lambda i: (i, 0, 0)),
        ],
        out_specs=pl.BlockSpec((G, d), lambda i: (0, 0)),
        out_shape=jax.ShapeDtypeStruct((G, d), jnp.float32),
        scratch_shapes=[
            pltpu.VMEM((G, d), jnp.float32),
            pltpu.VMEM((G, 1), jnp.float32),
        ],
    )(a2, dv, h, b, batch3)


def kernel(x, edge_index, edge_attr, batch, W0, b0, W1, b1):
    n, d = x.shape
    e = edge_index.shape[1]
    G = 16
    epw = e // NW
    K = _pick_chunk(epw)
    nch = epw // K

    src3 = edge_index[0].reshape(NW, nch, K)
    dst3 = edge_index[1].reshape(NW, nch, K)
    wnw = edge_attr.reshape(NW, epw)

    degp = _sc_deg(dst3, wnw, n).reshape(NC, n, L)
    d0 = degp[0, :, 0:1]
    d1 = degp[1, :, 0:1]

    dh = d // 2
    ept = e // NS
    nch2 = ept // K
    src2 = edge_index[0].reshape(NS, nch2, K)
    dst2 = edge_index[1].reshape(NS, nch2, K)
    w2 = edge_attr.reshape(NS, ept)

    def _scatter_full(ph):
        return _sc_scatter2(ph, src2, dst2, w2).reshape(NC, n, dh)

    h0, ph0, dv = _tc_stage1(d0, d1, x, W0)
    acc1 = _scatter_full(ph0)
    h1, ph1 = _tc_stage2(acc1, dv, h0, b0.reshape(1, d), W1)
    acc2 = _scatter_full(ph1)

    batch3 = batch.reshape(n // _BR, 1, _BR)
    return _tc_pool(acc2, dv, h1, b1.reshape(1, d), batch3, G)


# speedup vs baseline: 15.3192x; 1.3301x over previous
import functools

import jax
import jax.numpy as jnp
from jax import lax
from jax.experimental import pallas as pl
from jax.experimental.pallas import tpu as pltpu
from jax.experimental.pallas import tpu_sc as plsc

NC = 2
NS = 16
NW = NC * NS
L = 16


def _pick_chunk(epw):
    for k in range(128, 7, -8):
        if epw % k == 0:
            return k
    raise ValueError(f"no valid chunk for {epw}")


def _sc_mesh():
    return plsc.VectorSubcoreMesh(
        core_axis_name="c", subcore_axis_name="s",
        num_cores=NC, num_subcores=NS)


def _sc_deg(dst3, w2, n):
    nw, nch, K = dst3.shape
    epw = w2.shape[1]
    rpt = n // NS
    zc = rpt // 5

    @functools.partial(
        pl.kernel,
        out_type=jax.ShapeDtypeStruct((NC, NS, rpt, L), jnp.float32),
        mesh=_sc_mesh(),
        compiler_params=pltpu.CompilerParams(use_tc_tiling_on_sc=False),
        scratch_types=[
            pltpu.VMEM((nch, K), jnp.int32),
            pltpu.VMEM((epw,), jnp.float32),
            pltpu.VMEM((K, L), jnp.float32),
            pltpu.VMEM((zc, L), jnp.float32),
            pltpu.VMEM_SHARED((n, L), jnp.float32),
        ],
    )
    def k(dst_hbm, w_hbm, out_hbm, dstv, wv, msg, zb, acc):
        cid = lax.axis_index("c")
        sid = lax.axis_index("s")
        wid = cid * NS + sid

        def zrow(i, c):
            zb[i, :] = jnp.zeros((L,), jnp.float32)
            return c
        lax.fori_loop(0, zc, zrow, 0)
        for t in range(5):
            pltpu.sync_copy(zb, acc.at[pl.ds(sid * rpt + t * zc, zc)])
        plsc.subcore_barrier()

        pltpu.sync_copy(dst_hbm.at[wid], dstv)
        pltpu.sync_copy(w_hbm.at[wid], wv)

        def chunk(ck, c):
            base = ck * K
            for g in range(K // L):
                wvec = wv[pl.ds(base + g * L, L)]
                for i2 in range(L):
                    msg[g * L + i2, :] = jnp.full((L,), 1.0, jnp.float32) * wvec[i2]
            pltpu.sync_copy(msg, acc.at[dstv.at[ck]], add=True)
            return c
        lax.fori_loop(0, nch, chunk, 0)
        plsc.subcore_barrier()

        pltpu.sync_copy(acc.at[pl.ds(sid * rpt, rpt)], out_hbm.at[cid, sid])

    return k(dst3, w2)


def _sc_scatter2(ph, src2, dst2, w3):
    nc, n, dh = ph.shape
    ns, nch, K = src2.shape
    rpt = n // NS
    nv = dh // L
    NB = 5
    SL = 2
    zr = 25

    @functools.partial(
        pl.kernel,
        out_type=jax.ShapeDtypeStruct((NC, NS, rpt, dh), jnp.float32),
        mesh=_sc_mesh(),
        compiler_params=pltpu.CompilerParams(use_tc_tiling_on_sc=False),
        scratch_types=[
            pltpu.VMEM((nch, K), jnp.int32),
            pltpu.VMEM((nch, K), jnp.int32),
            pltpu.VMEM((NB, K), jnp.float32),
            pltpu.VMEM((NB, K, dh), jnp.float32),
            pltpu.VMEM((zr, dh), jnp.float32),
            pltpu.VMEM_SHARED((n, dh), jnp.float32),
            pltpu.SemaphoreType.DMA((NB,)),
            pltpu.SemaphoreType.DMA((NB,)),
            pltpu.SemaphoreType.DMA((NB,)),
        ],
    )
    def k(ph_hbm, src_hbm, dst_hbm, w_hbm, out_hbm,
          srcv, dstv, wring, ring, zb, acc, semg, semw, sems):
        cid = lax.axis_index("c")
        sid = lax.axis_index("s")

        pltpu.sync_copy(src_hbm.at[sid], srcv)
        pltpu.sync_copy(dst_hbm.at[sid], dstv)

        def zrow(i, c):
            for j in range(nv):
                zb[i, pl.ds(j * L, L)] = jnp.zeros((L,), jnp.float32)
            return c
        lax.fori_loop(0, zr, zrow, 0)
        for t in range(rpt // zr):
            pltpu.sync_copy(zb, acc.at[pl.ds(sid * rpt + t * zr, zr)])

        def g_start(ck, b):
            pltpu.async_copy(ph_hbm.at[cid].at[srcv.at[ck]], ring.at[b],
                             semg.at[b])
            pltpu.async_copy(w_hbm.at[sid, ck], wring.at[b], semw.at[b])

        def g_wait(ck, b):
            pltpu.make_async_copy(ph_hbm.at[cid].at[srcv.at[ck]], ring.at[b],
                                  semg.at[b]).wait()
            pltpu.make_async_copy(w_hbm.at[sid, ck], wring.at[b],
                                  semw.at[b]).wait()

        def s_start(ck, b):
            pltpu.async_copy(ring.at[b], acc.at[dstv.at[ck]], sems.at[b],
                             add=True)

        def s_wait(ck, b):
            pltpu.make_async_copy(ring.at[b], acc.at[dstv.at[ck]],
                                  sems.at[b]).wait()

        for b in range(NB):
            g_start(b, b)
        plsc.subcore_barrier()

        def outer(g, c):
            for u in range(NB):
                ck = g * NB + u
                g_wait(ck, u)

                def grp(q, c2):
                    wvec = wring[u, pl.ds(q * L, L)]
                    for i2 in range(L):
                        i = q * L + i2
                        ws = wvec[i2]
                        for j in range(nv):
                            sl = pl.ds(j * L, L)
                            ring[u, i, sl] = ring[u, i, sl] * ws
                    return c2
                lax.fori_loop(0, K // L, grp, 0)
                s_start(ck, u)

                ck2 = ck - SL
                b2 = (u - SL) % NB

                @pl.when(ck2 >= 0)
                def _():
                    s_wait(ck2, b2)

                @pl.when((ck2 >= 0) & (ck2 + NB < nch))
                def _():
                    g_start(ck2 + NB, b2)
            return c
        lax.fori_loop(0, nch // NB, outer, 0)

        for j in range(SL):
            m = nch - SL + j
            s_wait(m, m % NB)
        plsc.subcore_barrier()

        pltpu.sync_copy(acc.at[pl.ds(sid * rpt, rpt)], out_hbm.at[cid, sid])

    return k(ph, src2, dst2, w3)


_BR = 1000


def _tc_stage1(d0, d1, x, W0):
    n, d = x.shape
    dh = d // 2
    g = n // _BR

    def body(d0r, d1r, xr, wr, hr, phr, dvr):
        deg = d0r[...] + d1r[...] + 2.0
        dv = lax.rsqrt(deg)
        h = jnp.dot(xr[...], wr[...], preferred_element_type=jnp.float32,
                    precision=lax.Precision.HIGHEST)
        hr[...] = h
        p = h * dv
        phr[0, :, :] = p[:, :dh]
        phr[1, :, :] = p[:, dh:]
        dvr[...] = dv

    return pl.pallas_call(
        body,
        grid=(g,),
        in_specs=[
            pl.BlockSpec((_BR, 1), lambda i: (i, 0)),
            pl.BlockSpec((_BR, 1), lambda i: (i, 0)),
            pl.BlockSpec((_BR, d), lambda i: (i, 0)),
            pl.BlockSpec((d, d), lambda i: (0, 0)),
        ],
        out_specs=[
            pl.BlockSpec((_BR, d), lambda i: (i, 0)),
            pl.BlockSpec((2, _BR, dh), lambda i: (0, i, 0)),
            pl.BlockSpec((_BR, 1), lambda i: (i, 0)),
        ],
        out_shape=[
            jax.ShapeDtypeStruct((n, d), jnp.float32),
            jax.ShapeDtypeStruct((2, n, dh), jnp.float32),
            jax.ShapeDtypeStruct((n, 1), jnp.float32),
        ],
    )(d0, d1, x, W0)


def _tc_stage2(a2, dv, h, b, W):
    n, d = h.shape
    dh = d // 2
    g = n // _BR

    def body(ar, dvr, hr, br, wr, h1r, p1r):
        dvb = dvr[...]
        af = jnp.concatenate([ar[0], ar[1]], axis=1)
        o = dvb * af + (2.0 * dvb * dvb) * hr[...] + br[...]
        h1 = jnp.dot(o, wr[...], preferred_element_type=jnp.float32,
                     precision=lax.Precision.HIGHEST)
        h1r[...] = h1
        p1 = h1 * dvb
        p1r[0, :, :] = p1[:, :dh]
        p1r[1, :, :] = p1[:, dh:]

    return pl.pallas_call(
        body,
        grid=(g,),
        in_specs=[
            pl.BlockSpec((2, _BR, dh), lambda i: (0, i, 0)),
            pl.BlockSpec((_BR, 1), lambda i: (i, 0)),
            pl.BlockSpec((_BR, d), lambda i: (i, 0)),
            pl.BlockSpec((1, d), lambda i: (0, 0)),
            pl.BlockSpec((d, d), lambda i: (0, 0)),
        ],
        out_specs=[
            pl.BlockSpec((_BR, d), lambda i: (i, 0)),
            pl.BlockSpec((2, _BR, dh), lambda i: (0, i, 0)),
        ],
        out_shape=[
            jax.ShapeDtypeStruct((n, d), jnp.float32),
            jax.ShapeDtypeStruct((2, n, dh), jnp.float32),
        ],
    )(a2, dv, h, b, W)


def _tc_pool(a2, dv, h, b, batch3, G):
    n, d = h.shape
    dh = d // 2
    g = n // _BR

    def body(ar, dvr, hr, br, btr, outr, sums, cnt):
        i = pl.program_id(0)

        @pl.when(i == 0)
        def _():
            sums[...] = jnp.zeros((G, d), jnp.float32)
            cnt[...] = jnp.zeros((G, 1), jnp.float32)

        dvb = dvr[...]
        af = jnp.concatenate([ar[0], ar[1]], axis=1)
        o = dvb * af + (2.0 * dvb * dvb) * hr[...] + br[...]
        bt = btr[...].reshape(1, _BR)
        gi = lax.broadcasted_iota(jnp.int32, (G, _BR), 0)
        oh = jnp.where(gi == bt, 1.0, 0.0).astype(jnp.float32)
        sums[...] += jnp.dot(oh, o, preferred_element_type=jnp.float32,
                             precision=lax.Precision.HIGHEST)
        cnt[...] += jnp.sum(oh, axis=1, keepdims=True)

        @pl.when(i == g - 1)
        def _():
            outr[...] = sums[...] / jnp.maximum(cnt[...], 1.0)

    return pl.pallas_call(
        body,
        grid=(g,),
        in_specs=[
            pl.BlockSpec((2, _BR, dh), lambda i: (0, i, 0)),
            pl.BlockSpec((_BR, 1), lambda i: (i, 0)),
            pl.BlockSpec((_BR, d), lambda i: (i, 0)),
            pl.BlockSpec((1, d), lambda i: (0, 0)),
            pl.BlockSpec((1, 1, _BR), lambda i: (i, 0, 0)),
        ],
        out_specs=pl.BlockSpec((G, d), lambda i: (0, 0)),
        out_shape=jax.ShapeDtypeStruct((G, d), jnp.float32),
        scratch_shapes=[
            pltpu.VMEM((G, d), jnp.float32),
            pltpu.VMEM((G, 1), jnp.float32),
        ],
    )(a2, dv, h, b, batch3)


def kernel(x, edge_index, edge_attr, batch, W0, b0, W1, b1):
    n, d = x.shape
    e = edge_index.shape[1]
    G = 16
    epw = e // NW
    K = _pick_chunk(epw)
    nch = epw // K

    src3 = edge_index[0].reshape(NW, nch, K)
    dst3 = edge_index[1].reshape(NW, nch, K)
    wnw = edge_attr.reshape(NW, epw)

    degp = _sc_deg(dst3, wnw, n).reshape(NC, n, L)
    d0 = degp[0, :, 0:1]
    d1 = degp[1, :, 0:1]

    dh = d // 2
    ept = e // NS
    real = next(k for k in range(128, 0, -1) if ept % k == 0)
    KP = ((real + 7) // 8) * 8
    nch2 = ept // real

    def _pad3(a, fill):
        a3 = a.reshape(NS, nch2, real)
        if KP == real:
            return a3
        return jnp.pad(a3, ((0, 0), (0, 0), (0, KP - real)),
                       constant_values=fill)

    src2 = _pad3(edge_index[0], 0)
    dst2 = _pad3(edge_index[1], 0)
    w3 = _pad3(edge_attr.reshape(-1), 0.0)

    def _scatter_full(ph):
        return _sc_scatter2(ph, src2, dst2, w3).reshape(NC, n, dh)

    h0, ph0, dv = _tc_stage1(d0, d1, x, W0)
    acc1 = _scatter_full(ph0)
    h1, ph1 = _tc_stage2(acc1, dv, h0, b0.reshape(1, d), W1)
    acc2 = _scatter_full(ph1)

    batch3 = batch.reshape(n // _BR, 1, _BR)
    return _tc_pool(acc2, dv, h1, b1.reshape(1, d), batch3, G)


# speedup vs baseline: 15.3736x; 1.0035x over previous
import functools

import jax
import jax.numpy as jnp
from jax import lax
from jax.experimental import pallas as pl
from jax.experimental.pallas import tpu as pltpu
from jax.experimental.pallas import tpu_sc as plsc

NC = 2
NS = 16
NW = NC * NS
L = 16


def _pick_chunk(epw):
    for k in range(128, 7, -8):
        if epw % k == 0:
            return k
    raise ValueError(f"no valid chunk for {epw}")


def _sc_mesh():
    return plsc.VectorSubcoreMesh(
        core_axis_name="c", subcore_axis_name="s",
        num_cores=NC, num_subcores=NS)


def _sc_deg(dst3, w2, n):
    nw, nch, K = dst3.shape
    epw = w2.shape[1]
    rpt = n // NS
    zc = rpt // 5

    @functools.partial(
        pl.kernel,
        out_type=jax.ShapeDtypeStruct((NC, NS, rpt, L), jnp.float32),
        mesh=_sc_mesh(),
        compiler_params=pltpu.CompilerParams(use_tc_tiling_on_sc=False),
        scratch_types=[
            pltpu.VMEM((nch, K), jnp.int32),
            pltpu.VMEM((epw,), jnp.float32),
            pltpu.VMEM((K, L), jnp.float32),
            pltpu.VMEM((zc, L), jnp.float32),
            pltpu.VMEM_SHARED((n, L), jnp.float32),
        ],
    )
    def k(dst_hbm, w_hbm, out_hbm, dstv, wv, msg, zb, acc):
        cid = lax.axis_index("c")
        sid = lax.axis_index("s")
        wid = cid * NS + sid

        def zrow(i, c):
            zb[i, :] = jnp.zeros((L,), jnp.float32)
            return c
        lax.fori_loop(0, zc, zrow, 0)
        for t in range(5):
            pltpu.sync_copy(zb, acc.at[pl.ds(sid * rpt + t * zc, zc)])
        plsc.subcore_barrier()

        pltpu.sync_copy(dst_hbm.at[wid], dstv)
        pltpu.sync_copy(w_hbm.at[wid], wv)

        def chunk(ck, c):
            base = ck * K
            for g in range(K // L):
                wvec = wv[pl.ds(base + g * L, L)]
                for i2 in range(L):
                    msg[g * L + i2, :] = jnp.full((L,), 1.0, jnp.float32) * wvec[i2]
            pltpu.sync_copy(msg, acc.at[dstv.at[ck]], add=True)
            return c
        lax.fori_loop(0, nch, chunk, 0)
        plsc.subcore_barrier()

        pltpu.sync_copy(acc.at[pl.ds(sid * rpt, rpt)], out_hbm.at[cid, sid])

    return k(dst3, w2)


def _sc_scatter2(ph, src2, dst2, w3):
    nc, n, dh = ph.shape
    ns, nch, K = src2.shape
    rpt = n // NS
    nv = dh // L
    NB = 5
    SL = 2
    zr = 25

    @functools.partial(
        pl.kernel,
        out_type=jax.ShapeDtypeStruct((NC, NS, rpt, dh), jnp.float32),
        mesh=_sc_mesh(),
        compiler_params=pltpu.CompilerParams(use_tc_tiling_on_sc=False),
        scratch_types=[
            pltpu.VMEM((nch, K), jnp.int32),
            pltpu.VMEM((nch, K), jnp.int32),
            pltpu.VMEM((NB, K), jnp.float32),
            pltpu.VMEM((NB, K, dh), jnp.float32),
            pltpu.VMEM((zr, dh), jnp.float32),
            pltpu.VMEM_SHARED((n, dh), jnp.float32),
            pltpu.SemaphoreType.DMA((NB,)),
            pltpu.SemaphoreType.DMA((NB,)),
            pltpu.SemaphoreType.DMA((NB,)),
        ],
    )
    def k(ph_hbm, src_hbm, dst_hbm, w_hbm, out_hbm,
          srcv, dstv, wring, ring, zb, acc, semg, semw, sems):
        cid = lax.axis_index("c")
        sid = lax.axis_index("s")

        pltpu.sync_copy(src_hbm.at[sid], srcv)
        pltpu.sync_copy(dst_hbm.at[sid], dstv)

        def zrow(i, c):
            for j in range(nv):
                zb[i, pl.ds(j * L, L)] = jnp.zeros((L,), jnp.float32)
            return c
        lax.fori_loop(0, zr, zrow, 0)
        for t in range(rpt // zr):
            pltpu.sync_copy(zb, acc.at[pl.ds(sid * rpt + t * zr, zr)])

        def g_start(ck, b):
            pltpu.async_copy(ph_hbm.at[cid].at[srcv.at[ck]], ring.at[b],
                             semg.at[b])
            pltpu.async_copy(w_hbm.at[sid, ck], wring.at[b], semw.at[b])

        def g_wait(ck, b):
            pltpu.make_async_copy(ph_hbm.at[cid].at[srcv.at[ck]], ring.at[b],
                                  semg.at[b]).wait()
            pltpu.make_async_copy(w_hbm.at[sid, ck], wring.at[b],
                                  semw.at[b]).wait()

        def s_start(ck, b):
            pltpu.async_copy(ring.at[b], acc.at[dstv.at[ck]], sems.at[b],
                             add=True)

        def s_wait(ck, b):
            pltpu.make_async_copy(ring.at[b], acc.at[dstv.at[ck]],
                                  sems.at[b]).wait()

        for b in range(NB):
            g_start(b, b)
        plsc.subcore_barrier()

        def outer(g, c):
            for u in range(NB):
                ck = g * NB + u
                g_wait(ck, u)

                @plsc.parallel_loop(0, K // L, unroll=2)
                def grp(q):
                    wvec = wring[u, pl.ds(q * L, L)]
                    for i2 in range(L):
                        i = q * L + i2
                        ws = wvec[i2]
                        for j in range(nv):
                            sl = pl.ds(j * L, L)
                            ring[u, i, sl] = ring[u, i, sl] * ws
                s_start(ck, u)

                ck2 = ck - SL
                b2 = (u - SL) % NB

                @pl.when(ck2 >= 0)
                def _():
                    s_wait(ck2, b2)

                @pl.when((ck2 >= 0) & (ck2 + NB < nch))
                def _():
                    g_start(ck2 + NB, b2)
            return c
        lax.fori_loop(0, nch // NB, outer, 0)

        for j in range(SL):
            m = nch - SL + j
            s_wait(m, m % NB)
        plsc.subcore_barrier()

        pltpu.sync_copy(acc.at[pl.ds(sid * rpt, rpt)], out_hbm.at[cid, sid])

    return k(ph, src2, dst2, w3)


_BR = 1000


def _tc_stage1(d0, d1, x, W0):
    n, d = x.shape
    dh = d // 2
    g = n // _BR

    def body(d0r, d1r, xr, wr, hr, phr, dvr):
        deg = d0r[...] + d1r[...] + 2.0
        dv = lax.rsqrt(deg)
        h = jnp.dot(xr[...], wr[...], preferred_element_type=jnp.float32,
                    precision=lax.Precision.HIGHEST)
        hr[...] = h
        p = h * dv
        phr[0, :, :] = p[:, :dh]
        phr[1, :, :] = p[:, dh:]
        dvr[...] = dv

    return pl.pallas_call(
        body,
        grid=(g,),
        in_specs=[
            pl.BlockSpec((_BR, 1), lambda i: (i, 0)),
            pl.BlockSpec((_BR, 1), lambda i: (i, 0)),
            pl.BlockSpec((_BR, d), lambda i: (i, 0)),
            pl.BlockSpec((d, d), lambda i: (0, 0)),
        ],
        out_specs=[
            pl.BlockSpec((_BR, d), lambda i: (i, 0)),
            pl.BlockSpec((2, _BR, dh), lambda i: (0, i, 0)),
            pl.BlockSpec((_BR, 1), lambda i: (i, 0)),
        ],
        out_shape=[
            jax.ShapeDtypeStruct((n, d), jnp.float32),
            jax.ShapeDtypeStruct((2, n, dh), jnp.float32),
            jax.ShapeDtypeStruct((n, 1), jnp.float32),
        ],
    )(d0, d1, x, W0)


def _tc_stage2(a2, dv, h, b, W):
    n, d = h.shape
    dh = d // 2
    g = n // _BR

    def body(ar, dvr, hr, br, wr, h1r, p1r):
        dvb = dvr[...]
        af = jnp.concatenate([ar[0], ar[1]], axis=1)
        o = dvb * af + (2.0 * dvb * dvb) * hr[...] + br[...]
        h1 = jnp.dot(o, wr[...], preferred_element_type=jnp.float32,
                     precision=lax.Precision.HIGHEST)
        h1r[...] = h1
        p1 = h1 * dvb
        p1r[0, :, :] = p1[:, :dh]
        p1r[1, :, :] = p1[:, dh:]

    return pl.pallas_call(
        body,
        grid=(g,),
        in_specs=[
            pl.BlockSpec((2, _BR, dh), lambda i: (0, i, 0)),
            pl.BlockSpec((_BR, 1), lambda i: (i, 0)),
            pl.BlockSpec((_BR, d), lambda i: (i, 0)),
            pl.BlockSpec((1, d), lambda i: (0, 0)),
            pl.BlockSpec((d, d), lambda i: (0, 0)),
        ],
        out_specs=[
            pl.BlockSpec((_BR, d), lambda i: (i, 0)),
            pl.BlockSpec((2, _BR, dh), lambda i: (0, i, 0)),
        ],
        out_shape=[
            jax.ShapeDtypeStruct((n, d), jnp.float32),
            jax.ShapeDtypeStruct((2, n, dh), jnp.float32),
        ],
    )(a2, dv, h, b, W)


def _tc_pool(a2, dv, h, b, batch3, G):
    n, d = h.shape
    dh = d // 2
    g = n // _BR

    def body(ar, dvr, hr, br, btr, outr, sums, cnt):
        i = pl.program_id(0)

        @pl.when(i == 0)
        def _():
            sums[...] = jnp.zeros((G, d), jnp.float32)
            cnt[...] = jnp.zeros((G, 1), jnp.float32)

        dvb = dvr[...]
        af = jnp.concatenate([ar[0], ar[1]], axis=1)
        o = dvb * af + (2.0 * dvb * dvb) * hr[...] + br[...]
        bt = btr[...].reshape(1, _BR)
        gi = lax.broadcasted_iota(jnp.int32, (G, _BR), 0)
        oh = jnp.where(gi == bt, 1.0, 0.0).astype(jnp.float32)
        sums[...] += jnp.dot(oh, o, preferred_element_type=jnp.float32,
                             precision=lax.Precision.HIGHEST)
        cnt[...] += jnp.sum(oh, axis=1, keepdims=True)

        @pl.when(i == g - 1)
        def _():
            outr[...] = sums[...] / jnp.maximum(cnt[...], 1.0)

    return pl.pallas_call(
        body,
        grid=(g,),
        in_specs=[
            pl.BlockSpec((2, _BR, dh), lambda i: (0, i, 0)),
            pl.BlockSpec((_BR, 1), lambda i: (i, 0)),
            pl.BlockSpec((_BR, d), lambda i: (i, 0)),
            pl.BlockSpec((1, d), lambda i: (0, 0)),
            pl.BlockSpec((1, 1, _BR), lambda i: (i, 0, 0)),
        ],
        out_specs=pl.BlockSpec((G, d), lambda i: (0, 0)),
        out_shape=jax.ShapeDtypeStruct((G, d), jnp.float32),
        scratch_shapes=[
            pltpu.VMEM((G, d), jnp.float32),
            pltpu.VMEM((G, 1), jnp.float32),
        ],
    )(a2, dv, h, b, batch3)


def kernel(x, edge_index, edge_attr, batch, W0, b0, W1, b1):
    n, d = x.shape
    e = edge_index.shape[1]
    G = 16
    epw = e // NW
    K = _pick_chunk(epw)
    nch = epw // K

    src3 = edge_index[0].reshape(NW, nch, K)
    dst3 = edge_index[1].reshape(NW, nch, K)
    wnw = edge_attr.reshape(NW, epw)

    degp = _sc_deg(dst3, wnw, n).reshape(NC, n, L)
    d0 = degp[0, :, 0:1]
    d1 = degp[1, :, 0:1]

    dh = d // 2
    ept = e // NS
    real = next(k for k in range(128, 0, -1) if ept % k == 0)
    KP = ((real + 7) // 8) * 8
    nch2 = ept // real

    def _pad3(a, fill):
        a3 = a.reshape(NS, nch2, real)
        if KP == real:
            return a3
        return jnp.pad(a3, ((0, 0), (0, 0), (0, KP - real)),
                       constant_values=fill)

    src2 = _pad3(edge_index[0], 0)
    dst2 = _pad3(edge_index[1], 0)
    w3 = _pad3(edge_attr.reshape(-1), 0.0)

    def _scatter_full(ph):
        return _sc_scatter2(ph, src2, dst2, w3).reshape(NC, n, dh)

    h0, ph0, dv = _tc_stage1(d0, d1, x, W0)
    acc1 = _scatter_full(ph0)
    h1, ph1 = _tc_stage2(acc1, dv, h0, b0.reshape(1, d), W1)
    acc2 = _scatter_full(ph1)

    batch3 = batch.reshape(n // _BR, 1, _BR)
    return _tc_pool(acc2, dv, h1, b1.reshape(1, d), batch3, G)


# speedup vs baseline: 15.4231x; 1.0032x over previous
import functools

import jax
import jax.numpy as jnp
from jax import lax
from jax.experimental import pallas as pl
from jax.experimental.pallas import tpu as pltpu
from jax.experimental.pallas import tpu_sc as plsc

NC = 2
NS = 16
NW = NC * NS
L = 16


def _pick_chunk(epw):
    for k in range(128, 7, -8):
        if epw % k == 0:
            return k
    raise ValueError(f"no valid chunk for {epw}")


def _sc_mesh():
    return plsc.VectorSubcoreMesh(
        core_axis_name="c", subcore_axis_name="s",
        num_cores=NC, num_subcores=NS)


def _sc_deg(dst3, w2, n):
    nw, nch, K = dst3.shape
    epw = w2.shape[1]
    rpt = n // NS
    zc = rpt // 5

    @functools.partial(
        pl.kernel,
        out_type=jax.ShapeDtypeStruct((NC, NS, rpt, L), jnp.float32),
        mesh=_sc_mesh(),
        compiler_params=pltpu.CompilerParams(use_tc_tiling_on_sc=False),
        scratch_types=[
            pltpu.VMEM((nch, K), jnp.int32),
            pltpu.VMEM((epw,), jnp.float32),
            pltpu.VMEM((K, L), jnp.float32),
            pltpu.VMEM((zc, L), jnp.float32),
            pltpu.VMEM_SHARED((n, L), jnp.float32),
        ],
    )
    def k(dst_hbm, w_hbm, out_hbm, dstv, wv, msg, zb, acc):
        cid = lax.axis_index("c")
        sid = lax.axis_index("s")
        wid = cid * NS + sid

        def zrow(i, c):
            zb[i, :] = jnp.zeros((L,), jnp.float32)
            return c
        lax.fori_loop(0, zc, zrow, 0)
        for t in range(5):
            pltpu.sync_copy(zb, acc.at[pl.ds(sid * rpt + t * zc, zc)])
        plsc.subcore_barrier()

        pltpu.sync_copy(dst_hbm.at[wid], dstv)
        pltpu.sync_copy(w_hbm.at[wid], wv)

        def chunk(ck, c):
            base = ck * K
            for g in range(K // L):
                wvec = wv[pl.ds(base + g * L, L)]
                for i2 in range(L):
                    msg[g * L + i2, :] = jnp.full((L,), 1.0, jnp.float32) * wvec[i2]
            pltpu.sync_copy(msg, acc.at[dstv.at[ck]], add=True)
            return c
        lax.fori_loop(0, nch, chunk, 0)
        plsc.subcore_barrier()

        pltpu.sync_copy(acc.at[pl.ds(sid * rpt, rpt)], out_hbm.at[cid, sid])

    return k(dst3, w2)


def _sc_scatter2(ph, src2, dst2, w3):
    nc, n, dh = ph.shape
    ns, nch, K = src2.shape
    rpt = n // NS
    nv = dh // L
    NB = 5
    SL = 2
    zr = 25

    @functools.partial(
        pl.kernel,
        out_type=jax.ShapeDtypeStruct((NC, NS, rpt, dh), jnp.float32),
        mesh=_sc_mesh(),
        compiler_params=pltpu.CompilerParams(use_tc_tiling_on_sc=False),
        scratch_types=[
            pltpu.VMEM((nch, K), jnp.int32),
            pltpu.VMEM((NB, K), jnp.int32),
            pltpu.VMEM((NB, K), jnp.float32),
            pltpu.VMEM((NB, K, dh), jnp.float32),
            pltpu.VMEM((zr, dh), jnp.float32),
            pltpu.VMEM_SHARED((n, dh), jnp.float32),
            pltpu.SemaphoreType.DMA((NB,)),
            pltpu.SemaphoreType.DMA((NB,)),
            pltpu.SemaphoreType.DMA((NB,)),
            pltpu.SemaphoreType.DMA((NB,)),
        ],
    )
    def k(ph_hbm, src_hbm, dst_hbm, w_hbm, out_hbm,
          srcv, dring, wring, ring, zb, acc, semg, semw, semd, sems):
        cid = lax.axis_index("c")
        sid = lax.axis_index("s")

        pltpu.sync_copy(src_hbm.at[sid], srcv)

        def zrow(i, c):
            for j in range(nv):
                zb[i, pl.ds(j * L, L)] = jnp.zeros((L,), jnp.float32)
            return c
        lax.fori_loop(0, zr, zrow, 0)
        for t in range(rpt // zr):
            pltpu.sync_copy(zb, acc.at[pl.ds(sid * rpt + t * zr, zr)])

        def g_start(ck, b):
            pltpu.async_copy(ph_hbm.at[cid].at[srcv.at[ck]], ring.at[b],
                             semg.at[b])
            pltpu.async_copy(w_hbm.at[sid, ck], wring.at[b], semw.at[b])
            pltpu.async_copy(dst_hbm.at[sid, ck], dring.at[b], semd.at[b])

        def g_wait(ck, b):
            pltpu.make_async_copy(ph_hbm.at[cid].at[srcv.at[ck]], ring.at[b],
                                  semg.at[b]).wait()
            pltpu.make_async_copy(w_hbm.at[sid, ck], wring.at[b],
                                  semw.at[b]).wait()
            pltpu.make_async_copy(dst_hbm.at[sid, ck], dring.at[b],
                                  semd.at[b]).wait()

        def s_start(ck, b):
            pltpu.async_copy(ring.at[b], acc.at[dring.at[b]], sems.at[b],
                             add=True)

        def s_wait(ck, b):
            pltpu.make_async_copy(ring.at[b], acc.at[dring.at[b]],
                                  sems.at[b]).wait()

        for b in range(NB):
            g_start(b, b)
        plsc.subcore_barrier()

        def outer(g, c):
            for u in range(NB):
                ck = g * NB + u
                g_wait(ck, u)

                @plsc.parallel_loop(0, K // L, unroll=2)
                def grp(q):
                    wvec = wring[u, pl.ds(q * L, L)]
                    for i2 in range(L):
                        i = q * L + i2
                        ws = wvec[i2]
                        for j in range(nv):
                            sl = pl.ds(j * L, L)
                            ring[u, i, sl] = ring[u, i, sl] * ws
                s_start(ck, u)

                ck2 = ck - SL
                b2 = (u - SL) % NB

                @pl.when(ck2 >= 0)
                def _():
                    s_wait(ck2, b2)

                @pl.when((ck2 >= 0) & (ck2 + NB < nch))
                def _():
                    g_start(ck2 + NB, b2)
            return c
        lax.fori_loop(0, nch // NB, outer, 0)

        for j in range(SL):
            m = nch - SL + j
            s_wait(m, m % NB)
        plsc.subcore_barrier()

        pltpu.sync_copy(acc.at[pl.ds(sid * rpt, rpt)], out_hbm.at[cid, sid])

    return k(ph, src2, dst2, w3)


_BR = 1000


def _tc_stage1(d0, d1, x, W0):
    n, d = x.shape
    dh = d // 2
    g = n // _BR

    def body(d0r, d1r, xr, wr, hr, phr, dvr):
        deg = d0r[...] + d1r[...] + 2.0
        dv = lax.rsqrt(deg)
        h = jnp.dot(xr[...], wr[...], preferred_element_type=jnp.float32,
                    precision=lax.Precision.HIGHEST)
        hr[...] = h
        p = h * dv
        phr[0, :, :] = p[:, :dh]
        phr[1, :, :] = p[:, dh:]
        dvr[...] = dv

    return pl.pallas_call(
        body,
        grid=(g,),
        in_specs=[
            pl.BlockSpec((_BR, 1), lambda i: (i, 0)),
            pl.BlockSpec((_BR, 1), lambda i: (i, 0)),
            pl.BlockSpec((_BR, d), lambda i: (i, 0)),
            pl.BlockSpec((d, d), lambda i: (0, 0)),
        ],
        out_specs=[
            pl.BlockSpec((_BR, d), lambda i: (i, 0)),
            pl.BlockSpec((2, _BR, dh), lambda i: (0, i, 0)),
            pl.BlockSpec((_BR, 1), lambda i: (i, 0)),
        ],
        out_shape=[
            jax.ShapeDtypeStruct((n, d), jnp.float32),
            jax.ShapeDtypeStruct((2, n, dh), jnp.float32),
            jax.ShapeDtypeStruct((n, 1), jnp.float32),
        ],
    )(d0, d1, x, W0)


def _tc_stage2(a2, dv, h, b, W):
    n, d = h.shape
    dh = d // 2
    g = n // _BR

    def body(ar, dvr, hr, br, wr, h1r, p1r):
        dvb = dvr[...]
        af = jnp.concatenate([ar[0], ar[1]], axis=1)
        o = dvb * af + (2.0 * dvb * dvb) * hr[...] + br[...]
        h1 = jnp.dot(o, wr[...], preferred_element_type=jnp.float32,
                     precision=lax.Precision.HIGHEST)
        h1r[...] = h1
        p1 = h1 * dvb
        p1r[0, :, :] = p1[:, :dh]
        p1r[1, :, :] = p1[:, dh:]

    return pl.pallas_call(
        body,
        grid=(g,),
        in_specs=[
            pl.BlockSpec((2, _BR, dh), lambda i: (0, i, 0)),
            pl.BlockSpec((_BR, 1), lambda i: (i, 0)),
            pl.BlockSpec((_BR, d), lambda i: (i, 0)),
            pl.BlockSpec((1, d), lambda i: (0, 0)),
            pl.BlockSpec((d, d), lambda i: (0, 0)),
        ],
        out_specs=[
            pl.BlockSpec((_BR, d), lambda i: (i, 0)),
            pl.BlockSpec((2, _BR, dh), lambda i: (0, i, 0)),
        ],
        out_shape=[
            jax.ShapeDtypeStruct((n, d), jnp.float32),
            jax.ShapeDtypeStruct((2, n, dh), jnp.float32),
        ],
    )(a2, dv, h, b, W)


def _tc_pool(a2, dv, h, b, batch3, G):
    n, d = h.shape
    dh = d // 2
    g = n // _BR

    def body(ar, dvr, hr, br, btr, outr, sums, cnt):
        i = pl.program_id(0)

        @pl.when(i == 0)
        def _():
            sums[...] = jnp.zeros((G, d), jnp.float32)
            cnt[...] = jnp.zeros((G, 1), jnp.float32)

        dvb = dvr[...]
        af = jnp.concatenate([ar[0], ar[1]], axis=1)
        o = dvb * af + (2.0 * dvb * dvb) * hr[...] + br[...]
        bt = btr[...].reshape(1, _BR)
        gi = lax.broadcasted_iota(jnp.int32, (G, _BR), 0)
        oh = jnp.where(gi == bt, 1.0, 0.0).astype(jnp.float32)
        sums[...] += jnp.dot(oh, o, preferred_element_type=jnp.float32,
                             precision=lax.Precision.HIGHEST)
        cnt[...] += jnp.sum(oh, axis=1, keepdims=True)

        @pl.when(i == g - 1)
        def _():
            outr[...] = sums[...] / jnp.maximum(cnt[...], 1.0)

    return pl.pallas_call(
        body,
        grid=(g,),
        in_specs=[
            pl.BlockSpec((2, _BR, dh), lambda i: (0, i, 0)),
            pl.BlockSpec((_BR, 1), lambda i: (i, 0)),
            pl.BlockSpec((_BR, d), lambda i: (i, 0)),
            pl.BlockSpec((1, d), lambda i: (0, 0)),
            pl.BlockSpec((1, 1, _BR), lambda i: (i, 0, 0)),
        ],
        out_specs=pl.BlockSpec((G, d), lambda i: (0, 0)),
        out_shape=jax.ShapeDtypeStruct((G, d), jnp.float32),
        scratch_shapes=[
            pltpu.VMEM((G, d), jnp.float32),
            pltpu.VMEM((G, 1), jnp.float32),
        ],
    )(a2, dv, h, b, batch3)


def kernel(x, edge_index, edge_attr, batch, W0, b0, W1, b1):
    n, d = x.shape
    e = edge_index.shape[1]
    G = 16
    epw = e // NW
    K = _pick_chunk(epw)
    nch = epw // K

    src3 = edge_index[0].reshape(NW, nch, K)
    dst3 = edge_index[1].reshape(NW, nch, K)
    wnw = edge_attr.reshape(NW, epw)

    degp = _sc_deg(dst3, wnw, n).reshape(NC, n, L)
    d0 = degp[0, :, 0:1]
    d1 = degp[1, :, 0:1]

    dh = d // 2
    ept = e // NS
    real = next(k for k in range(128, 0, -1) if ept % k == 0)
    KP = ((real + 7) // 8) * 8
    nch2 = ept // real

    def _pad3(a, fill):
        a3 = a.reshape(NS, nch2, real)
        if KP == real:
            return a3
        return jnp.pad(a3, ((0, 0), (0, 0), (0, KP - real)),
                       constant_values=fill)

    src2 = _pad3(edge_index[0], 0)
    dst2 = _pad3(edge_index[1], 0)
    w3 = _pad3(edge_attr.reshape(-1), 0.0)

    def _scatter_full(ph):
        return _sc_scatter2(ph, src2, dst2, w3).reshape(NC, n, dh)

    h0, ph0, dv = _tc_stage1(d0, d1, x, W0)
    acc1 = _scatter_full(ph0)
    h1, ph1 = _tc_stage2(acc1, dv, h0, b0.reshape(1, d), W1)
    acc2 = _scatter_full(ph1)

    batch3 = batch.reshape(n // _BR, 1, _BR)
    return _tc_pool(acc2, dv, h1, b1.reshape(1, d), batch3, G)
